# Initial kernel scaffold; baseline (speedup 1.0000x reference)
import jax
import jax.numpy as jnp
from jax.experimental import pallas as pl


def kernel(x, edge_index, params):
    raise NotImplementedError("write your pallas kernel here")


# speedup vs baseline: 2.7884x; 2.7884x over previous
import functools

import jax
import jax.numpy as jnp
from jax import lax
from jax.experimental import pallas as pl
from jax.experimental.pallas import tpu as pltpu
from jax.experimental.pallas import tpu_sc as plsc

N_NODES = 10000
N_EDGES = 160000
NPAD = 10240
NW = 32
NPW = NPAD // NW
CE = 8000
LS = 2048
CAP = 162304
GC = 64
BIG = 3.0e38
TCBLK = 512

_SC_PARAMS = pltpu.CompilerParams(
    use_tc_tiling_on_sc=False, needs_layout_passes=False)


def _wid():
    return lax.axis_index("s") * 2 + lax.axis_index("c")


def _partition_kernel(src_hbm, dst_hbm, srcl_hbm, ldstl_hbm, ngrp_hbm,
                      srcb, dstb, stg_s, stg_d, misc):
    w = _wid()
    lo = w * NPW
    base = w * CAP

    def chunk_body(ch, carry):
        off_stage, off_hbm = carry
        pltpu.sync_copy(src_hbm.at[pl.ds(ch * CE, CE)], srcb)
        pltpu.sync_copy(dst_hbm.at[pl.ds(ch * CE, CE)], dstb)

        def vec_body(i, c2):
            o_s, o_h = c2
            d = dstb[pl.ds(i * 16, 16)]
            s = srcb[pl.ds(i * 16, 16)]
            ld = d - lo
            m = (ld >= 0) & (ld < NPW)
            plsc.store_compressed(stg_s.at[pl.ds(o_s, 16)], s, mask=m)
            plsc.store_compressed(stg_d.at[pl.ds(o_s, 16)], ld, mask=m)
            cnt = plsc.all_reduce_population_count(m)[0]
            o_s = o_s + cnt
            do_flush = o_s >= LS

            @pl.when(do_flush)
            def _():
                hoff = pl.multiple_of(base + o_h, 256)
                pltpu.sync_copy(stg_s.at[pl.ds(0, LS)],
                                srcl_hbm.at[pl.ds(hoff, LS)])
                pltpu.sync_copy(stg_d.at[pl.ds(0, LS)],
                                ldstl_hbm.at[pl.ds(hoff, LS)])
                rem_s = stg_s[pl.ds(LS, 16)]
                rem_d = stg_d[pl.ds(LS, 16)]
                stg_s[pl.ds(0, 16)] = rem_s
                stg_d[pl.ds(0, 16)] = rem_d

            o_s = jnp.where(do_flush, o_s - LS, o_s)
            o_h = jnp.where(do_flush, o_h + LS, o_h)
            return (o_s, o_h)

        return lax.fori_loop(0, CE // 16, vec_body, (off_stage, off_hbm))

    off_stage, off_hbm = lax.fori_loop(
        0, N_EDGES // CE, chunk_body, (jnp.int32(0), jnp.int32(0)))

    pad_s = jnp.zeros((16,), jnp.int32)
    pad_d = jnp.full((16,), NPW, jnp.int32)
    full = pad_s == 0
    for k in range(9):
        plsc.store_compressed(stg_s.at[pl.ds(off_stage + k * 16, 16)],
                              pad_s, mask=full)
        plsc.store_compressed(stg_d.at[pl.ds(off_stage + k * 16, 16)],
                              pad_d, mask=full)
    hoff = pl.multiple_of(base + off_hbm, 256)
    pltpu.sync_copy(stg_s.at[pl.ds(0, LS)],
                    srcl_hbm.at[pl.ds(hoff, LS)])
    pltpu.sync_copy(stg_d.at[pl.ds(0, LS)],
                    ldstl_hbm.at[pl.ds(hoff, LS)])

    total = off_hbm + off_stage
    nch = ((total + 127) // 128) * (128 // GC)
    misc[pl.ds(0, 16)] = jnp.broadcast_to(nch, (16,)).astype(jnp.int32)
    pltpu.sync_copy(misc.at[pl.ds(0, 16)], ngrp_hbm.at[pl.ds(w * 16, 16)])


def _make_partition():
    mesh = plsc.VectorSubcoreMesh(core_axis_name="c", subcore_axis_name="s")
    return pl.kernel(
        _partition_kernel,
        out_type=(
            jax.ShapeDtypeStruct((NW * CAP,), jnp.int32),
            jax.ShapeDtypeStruct((NW * CAP,), jnp.int32),
            jax.ShapeDtypeStruct((NW * 16,), jnp.int32),
        ),
        mesh=mesh,
        compiler_params=_SC_PARAMS,
        scratch_types=[
            pltpu.VMEM((CE,), jnp.int32),
            pltpu.VMEM((CE,), jnp.int32),
            pltpu.VMEM((LS + 160,), jnp.int32),
            pltpu.VMEM((LS + 160,), jnp.int32),
            pltpu.VMEM((16,), jnp.int32),
        ],
    )


def _segmin_kernel(d, f_hbm, srcl_hbm, ldstl_hbm, ngrp_hbm, smin_hbm,
                   acc, rows0, rows1, idx0, idx1, ld0, ld1, nb, sem0, sem1):
    w = _wid()
    base_list = w * CAP
    base_node = w * NPW

    pltpu.sync_copy(ngrp_hbm.at[pl.ds(w * 16, 16)], nb)
    nch = nb[pl.ds(0, 16)][0]

    big = jnp.full((16,), BIG, jnp.float32)

    def init_row(r, _):
        for c in range(d // 16):
            acc[r, pl.ds(c * 16, 16)] = big
        return 0
    lax.fori_loop(0, NPW + 1, init_row, 0)

    def start(g, idx, ldb, rows, sem):
        loff = pl.multiple_of(base_list + g * GC, 64)
        pltpu.sync_copy(srcl_hbm.at[pl.ds(loff, GC)], idx)
        pltpu.sync_copy(ldstl_hbm.at[pl.ds(loff, GC)], ldb)
        return pltpu.async_copy(f_hbm.at[idx], rows, sem)

    def process(rows, ldb):
        def grp_body(t, _):
            ldvec = ldb[pl.ds(t * 16, 16)]
            for j in range(16):
                ld = ldvec[j]
                for c in range(d // 16):
                    cur = acc[ld, pl.ds(c * 16, 16)]
                    val = rows[t * 16 + j, pl.ds(c * 16, 16)]
                    acc[ld, pl.ds(c * 16, 16)] = jnp.minimum(cur, val)
            return 0
        lax.fori_loop(0, GC // 16, grp_body, 0)

    @pl.when(nch > 0)
    def _():
        start(0, idx0, ld0, rows0, sem0).wait()

        def pair_body(p, _):
            g0 = 2 * p
            @pl.when(g0 + 1 < nch)
            def _():
                start(g0 + 1, idx1, ld1, rows1, sem1)
            process(rows0, ld0)

            @pl.when(g0 + 1 < nch)
            def _():
                @pl.when(g0 + 2 < nch)
                def _():
                    start(g0 + 2, idx0, ld0, rows0, sem0)
                pltpu.make_async_copy(f_hbm.at[idx1], rows1, sem1).wait()
                process(rows1, ld1)

                @pl.when(g0 + 2 < nch)
                def _():
                    pltpu.make_async_copy(f_hbm.at[idx0], rows0, sem0).wait()
            return 0

        lax.fori_loop(0, (nch + 1) // 2, pair_body, 0)

    pltpu.sync_copy(acc.at[pl.ds(0, NPW)], smin_hbm.at[pl.ds(base_node, NPW)])


def _make_segmin(d):
    mesh = plsc.VectorSubcoreMesh(core_axis_name="c", subcore_axis_name="s")
    return pl.kernel(
        functools.partial(_segmin_kernel, d),
        out_type=jax.ShapeDtypeStruct((NPAD, d), jnp.float32),
        mesh=mesh,
        compiler_params=_SC_PARAMS,
        scratch_types=[
            pltpu.VMEM((NPW + 1, d), jnp.float32),
            pltpu.VMEM((GC, d), jnp.float32),
            pltpu.VMEM((GC, d), jnp.float32),
            pltpu.VMEM((GC,), jnp.int32),
            pltpu.VMEM((GC,), jnp.int32),
            pltpu.VMEM((GC,), jnp.int32),
            pltpu.VMEM((GC,), jnp.int32),
            pltpu.VMEM((16,), jnp.int32),
            pltpu.SemaphoreType.DMA,
            pltpu.SemaphoreType.DMA,
        ],
    )


def _mm(x, w):
    return jax.lax.dot_general(x, w, (((1,), (0,)), ((), ())),
                               preferred_element_type=jnp.float32)


def _stage_a_kernel(acts, x_ref, *refs):
    out_ref = refs[-1]
    h = x_ref[...]
    for i, act in enumerate(acts):
        w = refs[2 * i][...]
        b = refs[2 * i + 1][...]
        h = _mm(h, w) + b
        if act:
            h = jnp.maximum(h, 0.0)
    out_ref[...] = h


def _stage_a(x, wbs, acts):
    n = x.shape[0]
    dout = wbs[-1][0].shape[1]
    grid = n // TCBLK
    in_specs = [pl.BlockSpec((TCBLK, x.shape[1]), lambda i: (i, 0))]
    args = [x]
    for w, b in wbs:
        b2 = b.reshape(1, -1)
        in_specs.append(pl.BlockSpec(w.shape, lambda i: (0, 0)))
        in_specs.append(pl.BlockSpec(b2.shape, lambda i: (0, 0)))
        args.extend([w, b2])
    return pl.pallas_call(
        functools.partial(_stage_a_kernel, acts),
        grid=(grid,),
        in_specs=in_specs,
        out_specs=pl.BlockSpec((TCBLK, dout), lambda i: (i, 0)),
        out_shape=jax.ShapeDtypeStruct((n, dout), jnp.float32),
    )(*args)


def _stage_b_kernel(nhead, f_ref, smin_ref, *refs):
    out_ref = refs[-1]
    f = f_ref[...]
    smin = smin_ref[...]
    agg = jnp.where(smin < 1e37, f - smin, 0.0)
    a = jnp.tanh(_mm(agg, refs[0][...]) + refs[1][...])
    a = jnp.tanh(_mm(a, refs[2][...]) + refs[3][...])
    a = _mm(a, refs[4][...]) + refs[5][...]
    z = _mm(f, refs[6][...]) + _mm(a, refs[7][...]) + refs[8][...]
    z = jnp.maximum(z, 0.0)
    f2 = z + f
    h = jnp.maximum(_mm(f2, refs[9][...]) + refs[10][...], 0.0)
    h = jnp.maximum(_mm(h, refs[11][...]) + refs[12][...], 0.0)
    h = _mm(h, refs[13][...]) + refs[14][...]
    h = h + f2
    if nhead:
        h = jnp.tanh(_mm(h, refs[15][...]) + refs[16][...])
        h = jnp.maximum(_mm(h, refs[17][...]) + refs[18][...], 0.0)
        h = _mm(h, refs[19][...]) + refs[20][...]
    out_ref[...] = h


def _stage_b(f, smin, p, head=None):
    n, dout = f.shape
    args = [f, smin]
    wlist = [p['m2_w0'], p['m2_b0'], p['m2_w1'], p['m2_b1'], p['m2_w2'],
             p['m2_b2'], p['fc2_w'][:dout], p['fc2_w'][dout:], p['fc2_b'],
             p['m3_w0'], p['m3_b0'], p['m3_w1'], p['m3_b1'], p['m3_w2'],
             p['m3_b2']]
    nhead = 0
    if head is not None:
        wlist += list(head)
        nhead = len(head) // 2
    in_specs = [pl.BlockSpec((TCBLK, dout), lambda i: (i, 0)),
                pl.BlockSpec((TCBLK, dout), lambda i: (i, 0))]
    for k, wb in enumerate(wlist):
        if wb.ndim == 1:
            wb = wb.reshape(1, -1)
            wlist[k] = wb
        in_specs.append(pl.BlockSpec(wb.shape, lambda i: (0, 0)))
        args.append(wb)
    od = dout if head is None else head[-2].shape[1]
    return pl.pallas_call(
        functools.partial(_stage_b_kernel, nhead),
        grid=(n // TCBLK,),
        in_specs=in_specs,
        out_specs=pl.BlockSpec((TCBLK, od), lambda i: (i, 0)),
        out_shape=jax.ShapeDtypeStruct((n, od), jnp.float32),
    )(*args)


def kernel(x, edge_index, params):
    src = edge_index[0].astype(jnp.int32)
    dst = edge_index[1].astype(jnp.int32)
    xp = jnp.pad(x, ((0, NPAD - N_NODES), (0, 0)))

    srcl, ldstl, ngrp = _make_partition()(src, dst)

    p1, p2, p3 = params['k1'], params['k2'], params['k3']

    f1 = _stage_a(xp, [
        (params['emb_w'], params['emb_b']),
        (p1['m1_w0'], p1['m1_b0']), (p1['m1_w1'], p1['m1_b1']),
        (p1['m1_w2'], p1['m1_b2']), (p1['fc1_w'], p1['fc1_b'])],
        acts=[False, True, True, False, True])
    smin1 = _make_segmin(64)(f1, srcl, ldstl, ngrp)
    h1 = _stage_b(f1, smin1, p1)

    f2 = _stage_a(h1, [
        (p2['m1_w0'], p2['m1_b0']), (p2['m1_w1'], p2['m1_b1']),
        (p2['m1_w2'], p2['m1_b2']), (p2['fc1_w'], p2['fc1_b'])],
        acts=[True, True, False, True])
    smin2 = _make_segmin(128)(f2, srcl, ldstl, ngrp)
    h2 = _stage_b(f2, smin2, p2)

    f3 = _stage_a(h2, [
        (p3['m1_w0'], p3['m1_b0']), (p3['m1_w1'], p3['m1_b1']),
        (p3['m1_w2'], p3['m1_b2']), (p3['fc1_w'], p3['fc1_b'])],
        acts=[True, True, False, True])
    smin3 = _make_segmin(256)(f3, srcl, ldstl, ngrp)
    out = _stage_b(f3, smin3, p3, head=[
        params['head_w0'], params['head_b0'],
        params['head_w1'], params['head_b1'],
        params['head_w2'], params['head_b2']])

    return out[:N_NODES]


# speedup vs baseline: 2.7919x; 1.0013x over previous
import functools

import jax
import jax.numpy as jnp
from jax import lax
from jax.experimental import pallas as pl
from jax.experimental.pallas import tpu as pltpu
from jax.experimental.pallas import tpu_sc as plsc

N_NODES = 10000
N_EDGES = 160000
NPAD = 10240
NR = 16
NPW = NPAD // NR
E2 = N_EDGES // 2
CE = 8000
LS = 2048
CAP = 82304
GC = 64
BIG = 3.0e38
TCBLK = 512

_SC_PARAMS = pltpu.CompilerParams(
    use_tc_tiling_on_sc=False, needs_layout_passes=False)


def _partition_kernel(src_hbm, dst_hbm, srcl_hbm, ldstl_hbm, ngrp_hbm,
                      srcb, dstb, stg_s, stg_d, misc):
    w = lax.axis_index("s") * 2 + lax.axis_index("c")
    lo = (w // 2) * NPW
    ebase = (w % 2) * E2
    base = w * CAP

    def chunk_body(ch, carry):
        off_stage, off_hbm = carry
        eoff = pl.multiple_of(ebase + ch * CE, 256)
        pltpu.sync_copy(src_hbm.at[pl.ds(eoff, CE)], srcb)
        pltpu.sync_copy(dst_hbm.at[pl.ds(eoff, CE)], dstb)

        def vec_body(i, c2):
            o_s, o_h = c2
            d = dstb[pl.ds(i * 16, 16)]
            s = srcb[pl.ds(i * 16, 16)]
            ld = d - lo
            m = (ld >= 0) & (ld < NPW)
            plsc.store_compressed(stg_s.at[pl.ds(o_s, 16)], s, mask=m)
            plsc.store_compressed(stg_d.at[pl.ds(o_s, 16)], ld, mask=m)
            cnt = plsc.all_reduce_population_count(m)[0]
            o_s = o_s + cnt
            do_flush = o_s >= LS

            @pl.when(do_flush)
            def _():
                hoff = pl.multiple_of(base + o_h, 256)
                pltpu.sync_copy(stg_s.at[pl.ds(0, LS)],
                                srcl_hbm.at[pl.ds(hoff, LS)])
                pltpu.sync_copy(stg_d.at[pl.ds(0, LS)],
                                ldstl_hbm.at[pl.ds(hoff, LS)])
                rem_s = stg_s[pl.ds(LS, 16)]
                rem_d = stg_d[pl.ds(LS, 16)]
                stg_s[pl.ds(0, 16)] = rem_s
                stg_d[pl.ds(0, 16)] = rem_d

            o_s = jnp.where(do_flush, o_s - LS, o_s)
            o_h = jnp.where(do_flush, o_h + LS, o_h)
            return (o_s, o_h)

        return lax.fori_loop(0, CE // 16, vec_body, (off_stage, off_hbm))

    off_stage, off_hbm = lax.fori_loop(
        0, E2 // CE, chunk_body, (jnp.int32(0), jnp.int32(0)))

    pad_s = jnp.zeros((16,), jnp.int32)
    pad_d = jnp.full((16,), NPW, jnp.int32)
    full = pad_s == 0
    for k in range(9):
        plsc.store_compressed(stg_s.at[pl.ds(off_stage + k * 16, 16)],
                              pad_s, mask=full)
        plsc.store_compressed(stg_d.at[pl.ds(off_stage + k * 16, 16)],
                              pad_d, mask=full)
    hoff = pl.multiple_of(base + off_hbm, 256)
    pltpu.sync_copy(stg_s.at[pl.ds(0, LS)],
                    srcl_hbm.at[pl.ds(hoff, LS)])
    pltpu.sync_copy(stg_d.at[pl.ds(0, LS)],
                    ldstl_hbm.at[pl.ds(hoff, LS)])

    total = off_hbm + off_stage
    nch = ((total + 127) // 128) * (128 // GC)
    misc[pl.ds(0, 16)] = jnp.broadcast_to(nch, (16,)).astype(jnp.int32)
    pltpu.sync_copy(misc.at[pl.ds(0, 16)], ngrp_hbm.at[pl.ds(w * 16, 16)])


def _make_partition():
    mesh = plsc.VectorSubcoreMesh(core_axis_name="c", subcore_axis_name="s")
    return pl.kernel(
        _partition_kernel,
        out_type=(
            jax.ShapeDtypeStruct((32 * CAP,), jnp.int32),
            jax.ShapeDtypeStruct((32 * CAP,), jnp.int32),
            jax.ShapeDtypeStruct((32 * 16,), jnp.int32),
        ),
        mesh=mesh,
        compiler_params=_SC_PARAMS,
        scratch_types=[
            pltpu.VMEM((CE,), jnp.int32),
            pltpu.VMEM((CE,), jnp.int32),
            pltpu.VMEM((LS + 160,), jnp.int32),
            pltpu.VMEM((LS + 160,), jnp.int32),
            pltpu.VMEM((16,), jnp.int32),
        ],
    )


def _segmin_kernel(dh, f3_hbm, srcl_hbm, ldstl_hbm, ngrp_hbm, smin_hbm,
                   acc, rows0, rows1, idx0, idx1, ld0, ld1, nb, sem0, sem1):
    c = lax.axis_index("c")
    s = lax.axis_index("s")
    fh = f3_hbm.at[c]

    big = jnp.full((16,), BIG, jnp.float32)

    def init_row(r, _):
        for k in range(dh // 16):
            acc[r, pl.ds(k * 16, 16)] = big
        return 0
    lax.fori_loop(0, NPW + 1, init_row, 0)

    def do_list(l):
        base_list = l * CAP
        pltpu.sync_copy(ngrp_hbm.at[pl.ds(l * 16, 16)], nb)
        nch = nb[pl.ds(0, 16)][0]

        def start(g, idx, ldb, rows, sem):
            loff = pl.multiple_of(base_list + g * GC, 64)
            pltpu.sync_copy(srcl_hbm.at[pl.ds(loff, GC)], idx)
            pltpu.sync_copy(ldstl_hbm.at[pl.ds(loff, GC)], ldb)
            return pltpu.async_copy(fh.at[idx], rows, sem)

        def process(rows, ldb):
            def grp_body(t, _):
                ldvec = ldb[pl.ds(t * 16, 16)]
                for j in range(16):
                    ld = ldvec[j]
                    for k in range(dh // 16):
                        cur = acc[ld, pl.ds(k * 16, 16)]
                        val = rows[t * 16 + j, pl.ds(k * 16, 16)]
                        acc[ld, pl.ds(k * 16, 16)] = jnp.minimum(cur, val)
                return 0
            lax.fori_loop(0, GC // 16, grp_body, 0)

        @pl.when(nch > 0)
        def _():
            start(0, idx0, ld0, rows0, sem0).wait()

            def pair_body(p, _):
                g0 = 2 * p

                @pl.when(g0 + 1 < nch)
                def _():
                    start(g0 + 1, idx1, ld1, rows1, sem1)
                process(rows0, ld0)

                @pl.when(g0 + 1 < nch)
                def _():
                    @pl.when(g0 + 2 < nch)
                    def _():
                        start(g0 + 2, idx0, ld0, rows0, sem0)
                    pltpu.make_async_copy(fh.at[idx1], rows1, sem1).wait()
                    process(rows1, ld1)

                    @pl.when(g0 + 2 < nch)
                    def _():
                        pltpu.make_async_copy(fh.at[idx0], rows0, sem0).wait()
                return 0

            lax.fori_loop(0, (nch + 1) // 2, pair_body, 0)

    do_list(s * 2)
    do_list(s * 2 + 1)

    pltpu.sync_copy(acc.at[pl.ds(0, NPW)],
                    smin_hbm.at[c].at[pl.ds(s * NPW, NPW)])


def _make_segmin(dh):
    mesh = plsc.VectorSubcoreMesh(core_axis_name="c", subcore_axis_name="s")
    return pl.kernel(
        functools.partial(_segmin_kernel, dh),
        out_type=jax.ShapeDtypeStruct((2, NPAD, dh), jnp.float32),
        mesh=mesh,
        compiler_params=_SC_PARAMS,
        scratch_types=[
            pltpu.VMEM((NPW + 1, dh), jnp.float32),
            pltpu.VMEM((GC, dh), jnp.float32),
            pltpu.VMEM((GC, dh), jnp.float32),
            pltpu.VMEM((GC,), jnp.int32),
            pltpu.VMEM((GC,), jnp.int32),
            pltpu.VMEM((GC,), jnp.int32),
            pltpu.VMEM((GC,), jnp.int32),
            pltpu.VMEM((16,), jnp.int32),
            pltpu.SemaphoreType.DMA,
            pltpu.SemaphoreType.DMA,
        ],
    )


def _mm(x, w):
    return jax.lax.dot_general(x, w, (((1,), (0,)), ((), ())),
                               preferred_element_type=jnp.float32)


def _stage_a_kernel(acts, dh, x_ref, *refs):
    out_ref = refs[-1]
    h = x_ref[...]
    for i, act in enumerate(acts):
        w = refs[2 * i][...]
        b = refs[2 * i + 1][...]
        h = _mm(h, w) + b
        if act:
            h = jnp.maximum(h, 0.0)
    out_ref[0] = h[:, :dh]
    out_ref[1] = h[:, dh:]


def _stage_a(x, wbs, acts):
    n = x.shape[0]
    dout = wbs[-1][0].shape[1]
    dh = dout // 2
    grid = n // TCBLK
    in_specs = [pl.BlockSpec((TCBLK, x.shape[1]), lambda i: (i, 0))]
    args = [x]
    for w, b in wbs:
        b2 = b.reshape(1, -1)
        in_specs.append(pl.BlockSpec(w.shape, lambda i: (0, 0)))
        in_specs.append(pl.BlockSpec(b2.shape, lambda i: (0, 0)))
        args.extend([w, b2])
    return pl.pallas_call(
        functools.partial(_stage_a_kernel, acts, dh),
        grid=(grid,),
        in_specs=in_specs,
        out_specs=pl.BlockSpec((2, TCBLK, dh), lambda i: (0, i, 0)),
        out_shape=jax.ShapeDtypeStruct((2, n, dh), jnp.float32),
    )(*args)


def _stage_b_kernel(nhead, f_ref, smin_ref, *refs):
    out_ref = refs[-1]
    f = jnp.concatenate([f_ref[0], f_ref[1]], axis=1)
    smin = jnp.concatenate([smin_ref[0], smin_ref[1]], axis=1)
    agg = jnp.where(smin < 1e37, f - smin, 0.0)
    a = jnp.tanh(_mm(agg, refs[0][...]) + refs[1][...])
    a = jnp.tanh(_mm(a, refs[2][...]) + refs[3][...])
    a = _mm(a, refs[4][...]) + refs[5][...]
    z = _mm(f, refs[6][...]) + _mm(a, refs[7][...]) + refs[8][...]
    z = jnp.maximum(z, 0.0)
    f2 = z + f
    h = jnp.maximum(_mm(f2, refs[9][...]) + refs[10][...], 0.0)
    h = jnp.maximum(_mm(h, refs[11][...]) + refs[12][...], 0.0)
    h = _mm(h, refs[13][...]) + refs[14][...]
    h = h + f2
    if nhead:
        h = jnp.tanh(_mm(h, refs[15][...]) + refs[16][...])
        h = jnp.maximum(_mm(h, refs[17][...]) + refs[18][...], 0.0)
        h = _mm(h, refs[19][...]) + refs[20][...]
    out_ref[...] = h


def _stage_b(f3, smin3, p, head=None):
    _, n, dh = f3.shape
    dout = 2 * dh
    args = [f3, smin3]
    wlist = [p['m2_w0'], p['m2_b0'], p['m2_w1'], p['m2_b1'], p['m2_w2'],
             p['m2_b2'], p['fc2_w'][:dout], p['fc2_w'][dout:], p['fc2_b'],
             p['m3_w0'], p['m3_b0'], p['m3_w1'], p['m3_b1'], p['m3_w2'],
             p['m3_b2']]
    nhead = 0
    if head is not None:
        wlist += list(head)
        nhead = len(head) // 2
    in_specs = [pl.BlockSpec((2, TCBLK, dh), lambda i: (0, i, 0)),
                pl.BlockSpec((2, TCBLK, dh), lambda i: (0, i, 0))]
    for k, wb in enumerate(wlist):
        if wb.ndim == 1:
            wb = wb.reshape(1, -1)
        in_specs.append(pl.BlockSpec(wb.shape, lambda i: (0, 0)))
        args.append(wb)
    od = dout if head is None else head[-2].shape[1]
    return pl.pallas_call(
        functools.partial(_stage_b_kernel, nhead),
        grid=(n // TCBLK,),
        in_specs=in_specs,
        out_specs=pl.BlockSpec((TCBLK, od), lambda i: (i, 0)),
        out_shape=jax.ShapeDtypeStruct((n, od), jnp.float32),
    )(*args)


def kernel(x, edge_index, params):
    src = edge_index[0].astype(jnp.int32)
    dst = edge_index[1].astype(jnp.int32)
    xp = jnp.pad(x, ((0, NPAD - N_NODES), (0, 0)))

    srcl, ldstl, ngrp = _make_partition()(src, dst)

    p1, p2, p3 = params['k1'], params['k2'], params['k3']

    f1 = _stage_a(xp, [
        (params['emb_w'], params['emb_b']),
        (p1['m1_w0'], p1['m1_b0']), (p1['m1_w1'], p1['m1_b1']),
        (p1['m1_w2'], p1['m1_b2']), (p1['fc1_w'], p1['fc1_b'])],
        acts=[False, True, True, False, True])
    smin1 = _make_segmin(32)(f1, srcl, ldstl, ngrp)
    h1 = _stage_b(f1, smin1, p1)

    f2 = _stage_a(h1, [
        (p2['m1_w0'], p2['m1_b0']), (p2['m1_w1'], p2['m1_b1']),
        (p2['m1_w2'], p2['m1_b2']), (p2['fc1_w'], p2['fc1_b'])],
        acts=[True, True, False, True])
    smin2 = _make_segmin(64)(f2, srcl, ldstl, ngrp)
    h2 = _stage_b(f2, smin2, p2)

    f3 = _stage_a(h2, [
        (p3['m1_w0'], p3['m1_b0']), (p3['m1_w1'], p3['m1_b1']),
        (p3['m1_w2'], p3['m1_b2']), (p3['fc1_w'], p3['fc1_b'])],
        acts=[True, True, False, True])
    smin3 = _make_segmin(128)(f3, srcl, ldstl, ngrp)
    out = _stage_b(f3, smin3, p3, head=[
        params['head_w0'], params['head_b0'],
        params['head_w1'], params['head_b1'],
        params['head_w2'], params['head_b2']])

    return out[:N_NODES]
